# Initial kernel scaffold; baseline (speedup 1.0000x reference)
import jax
import jax.numpy as jnp
from jax.experimental import pallas as pl


def kernel(x, edge_index, Ws0, bs0, Wn0, bn0, Ws1, bs1, Wn1, bn1, Ws2, bs2, Wn2, bn2, Wc, bc):
    raise NotImplementedError("write your pallas kernel here")


# speedup vs baseline: 3.9034x; 3.9034x over previous
import functools

import jax
import jax.numpy as jnp
from jax import lax
from jax.experimental import pallas as pl
from jax.experimental.pallas import tpu as pltpu
from jax.experimental.pallas import tpu_sc as plsc

N = 10000
E = 320000
D_IN = 128
D_H = 256
N_CLASSES = 10

NTILES = 16
N_PAD = 10240
ROWS_PER_TILE = N_PAD // NTILES
KB = 128
N_BATCHES = -(-E // (NTILES * KB))
EDGES_PER_TILE = N_BATCHES * KB
E_PAD = EDGES_PER_TILE * NTILES

BN = 400
GRID = N // BN


@functools.lru_cache(maxsize=None)
def _make_sc_agg(dh):
    mesh = plsc.VectorSubcoreMesh(core_axis_name="c", subcore_axis_name="s",
                                  num_cores=2, num_subcores=NTILES)

    out_type = [jax.ShapeDtypeStruct((N_PAD, dh), jnp.float32),
                jax.ShapeDtypeStruct((N_PAD, dh), jnp.float32)]
    scratch = [
        pltpu.VMEM((KB,), jnp.int32),
        pltpu.VMEM((KB,), jnp.int32),
        pltpu.VMEM((KB, dh), jnp.float32),
        pltpu.VMEM_SHARED((N_PAD, dh), jnp.float32),
        pltpu.SemaphoreType.DMA,
    ]

    @functools.partial(pl.kernel, mesh=mesh, out_type=out_type,
                       scratch_types=scratch)
    def k(hlo, hhi, src, dst, zd, alo, ahi, srcv, dstv, rows, agg, sem):
        cid = lax.axis_index("c")
        sid = lax.axis_index("s")
        r0 = sid * ROWS_PER_TILE

        pltpu.sync_copy(zd.at[pl.ds(r0, ROWS_PER_TILE)],
                        agg.at[pl.ds(r0, ROWS_PER_TILE)])

        plsc.subcore_barrier()

        ebase = sid * EDGES_PER_TILE

        def edge_loop(h_half):
            def step(i, c):
                off = pl.multiple_of(ebase + i * KB, 8)
                pltpu.sync_copy(src.at[pl.ds(off, KB)], srcv)
                pltpu.sync_copy(dst.at[pl.ds(off, KB)], dstv)
                pltpu.async_copy(h_half.at[srcv], rows, sem).wait()
                pltpu.sync_copy(rows, agg.at[dstv], add=True)
                return c
            lax.fori_loop(0, N_BATCHES, step, 0)

        @pl.when(cid == 0)
        def _():
            edge_loop(hlo)

        @pl.when(cid == 1)
        def _():
            edge_loop(hhi)

        plsc.subcore_barrier()

        @pl.when(cid == 0)
        def _():
            pltpu.sync_copy(agg.at[pl.ds(r0, ROWS_PER_TILE)],
                            alo.at[pl.ds(r0, ROWS_PER_TILE)])

        @pl.when(cid == 1)
        def _():
            pltpu.sync_copy(agg.at[pl.ds(r0, ROWS_PER_TILE)],
                            ahi.at[pl.ds(r0, ROWS_PER_TILE)])

    return k


def _dot(a, b):
    return jnp.dot(a, b, preferred_element_type=jnp.float32)


def _row_spec(w):
    return pl.BlockSpec((BN, w), lambda i: (i, 0))


def _full_spec(r, c):
    return pl.BlockSpec((r, c), lambda i: (0, 0))


def _tc_layer0(x, alo, ahi, WsT, WnloT, WnhiT, b):
    def body(x_r, alo_r, ahi_r, ws_r, wnlo_r, wnhi_r, b_r, olo_r, ohi_r):
        hs = _dot(x_r[...], ws_r[...])
        an = _dot(alo_r[...], wnlo_r[...]) + _dot(ahi_r[...], wnhi_r[...])
        d = jnp.maximum(alo_r[...][:, 64:65], 1.0)
        o = jnp.maximum(hs + an / d + b_r[...], 0.0)
        olo_r[...] = o[:, :128]
        ohi_r[...] = o[:, 128:]

    return pl.pallas_call(
        body,
        grid=(GRID,),
        in_specs=[_row_spec(128), _row_spec(128), _row_spec(128),
                  _full_spec(128, 256), _full_spec(128, 256),
                  _full_spec(128, 256), _full_spec(1, 256)],
        out_specs=[_row_spec(128), _row_spec(128)],
        out_shape=[jax.ShapeDtypeStruct((N, 128), jnp.float32),
                   jax.ShapeDtypeStruct((N, 128), jnp.float32)],
    )(x, alo, ahi, WsT, WnloT, WnhiT, b)


def _tc_layer_mid(hlo, hhi, alo, ahi, deg, WsloT, WshiT, WnloT, WnhiT, b):
    def body(hlo_r, hhi_r, alo_r, ahi_r, deg_r, wslo_r, wshi_r, wnlo_r,
             wnhi_r, b_r, olo_r, ohi_r):
        hs = _dot(hlo_r[...], wslo_r[...]) + _dot(hhi_r[...], wshi_r[...])
        an = _dot(alo_r[...], wnlo_r[...]) + _dot(ahi_r[...], wnhi_r[...])
        d = jnp.maximum(deg_r[...][:, 64:65], 1.0)
        o = jnp.maximum(hs + an / d + b_r[...], 0.0)
        olo_r[...] = o[:, :128]
        ohi_r[...] = o[:, 128:]

    return pl.pallas_call(
        body,
        grid=(GRID,),
        in_specs=[_row_spec(128)] * 5 +
                 [_full_spec(128, 256)] * 4 + [_full_spec(1, 256)],
        out_specs=[_row_spec(128), _row_spec(128)],
        out_shape=[jax.ShapeDtypeStruct((N, 128), jnp.float32),
                   jax.ShapeDtypeStruct((N, 128), jnp.float32)],
    )(hlo, hhi, alo, ahi, deg, WsloT, WshiT, WnloT, WnhiT, b)


def _tc_layer_final(hlo, hhi, alo, ahi, deg, WsloT, WshiT, WnloT, WnhiT, b,
                    WcT, bc):
    def body(hlo_r, hhi_r, alo_r, ahi_r, deg_r, wslo_r, wshi_r, wnlo_r,
             wnhi_r, b_r, wct_r, bc_r, cs_r, y_r):
        i = pl.program_id(0)
        hs = _dot(hlo_r[...], wslo_r[...]) + _dot(hhi_r[...], wshi_r[...])
        an = _dot(alo_r[...], wnlo_r[...]) + _dot(ahi_r[...], wnhi_r[...])
        d = jnp.maximum(deg_r[...][:, 64:65], 1.0)
        o = jnp.maximum(hs + an / d + b_r[...], 0.0)
        csum = jnp.sum(o, axis=0, keepdims=True)

        @pl.when(i == 0)
        def _():
            cs_r[...] = csum

        @pl.when(i > 0)
        def _():
            cs_r[...] = cs_r[...] + csum

        @pl.when(i == GRID - 1)
        def _():
            hg = cs_r[...] * (1.0 / N)
            y_r[...] = _dot(hg, wct_r[...]) + bc_r[...]

    csum_out, y = pl.pallas_call(
        body,
        grid=(GRID,),
        in_specs=[_row_spec(128)] * 5 +
                 [_full_spec(128, 256)] * 4 + [_full_spec(1, 256)] +
                 [_full_spec(256, N_CLASSES), _full_spec(1, N_CLASSES)],
        out_specs=[_full_spec(1, 256), _full_spec(1, N_CLASSES)],
        out_shape=[jax.ShapeDtypeStruct((1, 256), jnp.float32),
                   jax.ShapeDtypeStruct((1, N_CLASSES), jnp.float32)],
    )(hlo, hhi, alo, ahi, deg, WsloT, WshiT, WnloT, WnhiT, b, WcT, bc)
    del csum_out
    return y


def kernel(x, edge_index, Ws0, bs0, Wn0, bn0, Ws1, bs1, Wn1, bn1,
           Ws2, bs2, Wn2, bn2, Wc, bc):
    n_pad = E_PAD - E
    src = jnp.concatenate([edge_index[0].astype(jnp.int32),
                           jnp.arange(n_pad, dtype=jnp.int32) % N])
    dst = jnp.concatenate([edge_index[1].astype(jnp.int32),
                           N + (jnp.arange(n_pad, dtype=jnp.int32) % (N_PAD - N))])

    onec = jnp.ones((N, 1), jnp.float32)
    zc = jnp.zeros((N, 63), jnp.float32)
    x_lo = jnp.concatenate([x[:, :64], onec, zc], axis=1)
    x_hi = jnp.concatenate([x[:, 64:], onec, zc], axis=1)

    z128 = jnp.zeros((N_PAD, 128), jnp.float32)

    zw = jnp.zeros((64, 256), jnp.float32)
    wn0lo = jnp.concatenate([Wn0[:, :64].T, zw], axis=0)
    wn0hi = jnp.concatenate([Wn0[:, 64:].T, zw], axis=0)

    alo0, ahi0 = _make_sc_agg(128)(x_lo, x_hi, src, dst, z128)
    h_lo, h_hi = _tc_layer0(
        x, alo0, ahi0, Ws0.T, wn0lo, wn0hi, (bs0 + bn0)[None, :])

    alo, ahi = _make_sc_agg(128)(h_lo, h_hi, src, dst, z128)
    h_lo, h_hi = _tc_layer_mid(
        h_lo, h_hi, alo, ahi, alo0, Ws1[:, :128].T, Ws1[:, 128:].T,
        Wn1[:, :128].T, Wn1[:, 128:].T, (bs1 + bn1)[None, :])

    alo, ahi = _make_sc_agg(128)(h_lo, h_hi, src, dst, z128)
    y = _tc_layer_final(
        h_lo, h_hi, alo, ahi, alo0, Ws2[:, :128].T, Ws2[:, 128:].T,
        Wn2[:, :128].T, Wn2[:, 128:].T, (bs2 + bn2)[None, :],
        Wc.T, bc[None, :])
    return y


# speedup vs baseline: 6.9215x; 1.7732x over previous
import functools

import jax
import jax.numpy as jnp
from jax import lax
from jax.experimental import pallas as pl
from jax.experimental.pallas import tpu as pltpu
from jax.experimental.pallas import tpu_sc as plsc

N = 10000
E = 320000
D_IN = 128
D_H = 256
N_CLASSES = 10

NTILES = 16
N_PAD = 10240
ROWS_PER_TILE = N_PAD // NTILES
KB = 128
CHUNK = 16
CHUNK2 = CHUNK // 2
N_CHUNKS = 10
N_BATCHES = CHUNK * N_CHUNKS
EDGES_PER_TILE = N_BATCHES * KB
E_PAD = EDGES_PER_TILE * NTILES

BN = 400
GRID = N // BN


@functools.lru_cache(maxsize=None)
def _make_sc_agg(dh):
    mesh = plsc.VectorSubcoreMesh(core_axis_name="c", subcore_axis_name="s",
                                  num_cores=2, num_subcores=NTILES)

    out_type = [jax.ShapeDtypeStruct((N_PAD, dh), jnp.float32),
                jax.ShapeDtypeStruct((N_PAD, dh), jnp.float32)]
    scratch = [
        pltpu.VMEM((2, CHUNK, KB), jnp.int32),
        pltpu.VMEM((2, CHUNK, KB), jnp.int32),
        pltpu.VMEM((2, KB, dh), jnp.float32),
        pltpu.VMEM_SHARED((N_PAD, dh), jnp.float32),
        pltpu.SemaphoreType.DMA,
        pltpu.SemaphoreType.DMA,
        pltpu.SemaphoreType.DMA,
    ]

    @functools.partial(pl.kernel, mesh=mesh, out_type=out_type,
                       scratch_types=scratch)
    def k(hlo, hhi, src2, dst2, zd, alo, ahi, srcs, dsts, rows, agg,
          gsem, ssem, isem):
        cid = lax.axis_index("c")
        sid = lax.axis_index("s")
        r0 = sid * ROWS_PER_TILE
        bbase = sid * N_BATCHES

        def load_chunk(q, p):
            off = pl.multiple_of(bbase + q * CHUNK, 8)
            pltpu.async_copy(src2.at[pl.ds(off, CHUNK)], srcs.at[p], isem)
            pltpu.async_copy(dst2.at[pl.ds(off, CHUNK)], dsts.at[p], isem)

        def wait_chunk():
            pltpu.make_async_copy(src2.at[pl.ds(0, CHUNK)], srcs.at[0],
                                  isem).wait()
            pltpu.make_async_copy(dst2.at[pl.ds(0, CHUNK)], dsts.at[0],
                                  isem).wait()

        load_chunk(0, 0)
        pltpu.sync_copy(zd.at[pl.ds(r0, ROWS_PER_TILE)],
                        agg.at[pl.ds(r0, ROWS_PER_TILE)])
        wait_chunk()

        plsc.subcore_barrier()

        def edge_loop(h_half):
            def g(p, i, b):
                pltpu.async_copy(h_half.at[srcs.at[p, i]], rows.at[b], gsem)

            def s(p, i, b):
                pltpu.async_copy(rows.at[b], agg.at[dsts.at[p, i]], ssem,
                                 add=True)

            def wait_g():
                pltpu.make_async_copy(h_half.at[srcs.at[0, 0]], rows.at[0],
                                      gsem).wait()

            def wait_s():
                pltpu.make_async_copy(rows.at[0], agg.at[dsts.at[0, 0]],
                                      ssem).wait()

            def chunk_body(q, c):
                p = lax.rem(q, 2)

                @pl.when(q < N_CHUNKS - 1)
                def _():
                    load_chunk(q + 1, 1 - p)

                g(p, 0, 0)

                def body(j, c2):
                    i0 = j * 2
                    wait_g()
                    g(p, i0 + 1, 1)
                    s(p, i0, 0)
                    wait_g()
                    wait_s()

                    @pl.when(j < CHUNK2 - 1)
                    def _():
                        g(p, i0 + 2, 0)

                    s(p, i0 + 1, 1)
                    wait_s()
                    return c2

                lax.fori_loop(0, CHUNK2, body, 0)

                @pl.when(q < N_CHUNKS - 1)
                def _():
                    wait_chunk()
                return c

            lax.fori_loop(0, N_CHUNKS, chunk_body, 0)

        @pl.when(cid == 0)
        def _():
            edge_loop(hlo)

        @pl.when(cid == 1)
        def _():
            edge_loop(hhi)

        plsc.subcore_barrier()

        @pl.when(cid == 0)
        def _():
            pltpu.sync_copy(agg.at[pl.ds(r0, ROWS_PER_TILE)],
                            alo.at[pl.ds(r0, ROWS_PER_TILE)])

        @pl.when(cid == 1)
        def _():
            pltpu.sync_copy(agg.at[pl.ds(r0, ROWS_PER_TILE)],
                            ahi.at[pl.ds(r0, ROWS_PER_TILE)])

    return k


def _dot(a, b):
    return jnp.dot(a, b, preferred_element_type=jnp.float32)


def _row_spec(w):
    return pl.BlockSpec((BN, w), lambda i: (i, 0))


def _full_spec(r, c):
    return pl.BlockSpec((r, c), lambda i: (0, 0))


def _tc_layer0(x, alo, ahi, WsT, WnloT, WnhiT, b):
    def body(x_r, alo_r, ahi_r, ws_r, wnlo_r, wnhi_r, b_r, olo_r, ohi_r):
        hs = _dot(x_r[...], ws_r[...])
        an = _dot(alo_r[...], wnlo_r[...]) + _dot(ahi_r[...], wnhi_r[...])
        d = jnp.maximum(alo_r[...][:, 64:65], 1.0)
        o = jnp.maximum(hs + an / d + b_r[...], 0.0)
        olo_r[...] = o[:, :128]
        ohi_r[...] = o[:, 128:]

    return pl.pallas_call(
        body,
        grid=(GRID,),
        in_specs=[_row_spec(128), _row_spec(128), _row_spec(128),
                  _full_spec(128, 256), _full_spec(128, 256),
                  _full_spec(128, 256), _full_spec(1, 256)],
        out_specs=[_row_spec(128), _row_spec(128)],
        out_shape=[jax.ShapeDtypeStruct((N, 128), jnp.float32),
                   jax.ShapeDtypeStruct((N, 128), jnp.float32)],
    )(x, alo, ahi, WsT, WnloT, WnhiT, b)


def _tc_layer_mid(hlo, hhi, alo, ahi, deg, WsloT, WshiT, WnloT, WnhiT, b):
    def body(hlo_r, hhi_r, alo_r, ahi_r, deg_r, wslo_r, wshi_r, wnlo_r,
             wnhi_r, b_r, olo_r, ohi_r):
        hs = _dot(hlo_r[...], wslo_r[...]) + _dot(hhi_r[...], wshi_r[...])
        an = _dot(alo_r[...], wnlo_r[...]) + _dot(ahi_r[...], wnhi_r[...])
        d = jnp.maximum(deg_r[...][:, 64:65], 1.0)
        o = jnp.maximum(hs + an / d + b_r[...], 0.0)
        olo_r[...] = o[:, :128]
        ohi_r[...] = o[:, 128:]

    return pl.pallas_call(
        body,
        grid=(GRID,),
        in_specs=[_row_spec(128)] * 5 +
                 [_full_spec(128, 256)] * 4 + [_full_spec(1, 256)],
        out_specs=[_row_spec(128), _row_spec(128)],
        out_shape=[jax.ShapeDtypeStruct((N, 128), jnp.float32),
                   jax.ShapeDtypeStruct((N, 128), jnp.float32)],
    )(hlo, hhi, alo, ahi, deg, WsloT, WshiT, WnloT, WnhiT, b)


def _tc_layer_final(hlo, hhi, alo, ahi, deg, WsloT, WshiT, WnloT, WnhiT, b,
                    WcT, bc):
    def body(hlo_r, hhi_r, alo_r, ahi_r, deg_r, wslo_r, wshi_r, wnlo_r,
             wnhi_r, b_r, wct_r, bc_r, cs_r, y_r):
        i = pl.program_id(0)
        hs = _dot(hlo_r[...], wslo_r[...]) + _dot(hhi_r[...], wshi_r[...])
        an = _dot(alo_r[...], wnlo_r[...]) + _dot(ahi_r[...], wnhi_r[...])
        d = jnp.maximum(deg_r[...][:, 64:65], 1.0)
        o = jnp.maximum(hs + an / d + b_r[...], 0.0)
        csum = jnp.sum(o, axis=0, keepdims=True)

        @pl.when(i == 0)
        def _():
            cs_r[...] = csum

        @pl.when(i > 0)
        def _():
            cs_r[...] = cs_r[...] + csum

        @pl.when(i == GRID - 1)
        def _():
            hg = cs_r[...] * (1.0 / N)
            y_r[...] = _dot(hg, wct_r[...]) + bc_r[...]

    csum_out, y = pl.pallas_call(
        body,
        grid=(GRID,),
        in_specs=[_row_spec(128)] * 5 +
                 [_full_spec(128, 256)] * 4 + [_full_spec(1, 256)] +
                 [_full_spec(256, N_CLASSES), _full_spec(1, N_CLASSES)],
        out_specs=[_full_spec(1, 256), _full_spec(1, N_CLASSES)],
        out_shape=[jax.ShapeDtypeStruct((1, 256), jnp.float32),
                   jax.ShapeDtypeStruct((1, N_CLASSES), jnp.float32)],
    )(hlo, hhi, alo, ahi, deg, WsloT, WshiT, WnloT, WnhiT, b, WcT, bc)
    del csum_out
    return y


def kernel(x, edge_index, Ws0, bs0, Wn0, bn0, Ws1, bs1, Wn1, bn1,
           Ws2, bs2, Wn2, bn2, Wc, bc):
    n_pad = E_PAD - E
    src = jnp.concatenate([edge_index[0].astype(jnp.int32),
                           jnp.arange(n_pad, dtype=jnp.int32) % N])
    dst = jnp.concatenate([edge_index[1].astype(jnp.int32),
                           N + (jnp.arange(n_pad, dtype=jnp.int32) % (N_PAD - N))])
    src = src.reshape(NTILES * N_BATCHES, KB)
    dst = dst.reshape(NTILES * N_BATCHES, KB)

    onec = jnp.ones((N, 1), jnp.float32)
    zc = jnp.zeros((N, 63), jnp.float32)
    x_lo = jnp.concatenate([x[:, :64], onec, zc], axis=1)
    x_hi = jnp.concatenate([x[:, 64:], onec, zc], axis=1)

    z128 = jnp.zeros((N_PAD, 128), jnp.float32)

    zw = jnp.zeros((64, 256), jnp.float32)
    wn0lo = jnp.concatenate([Wn0[:, :64].T, zw], axis=0)
    wn0hi = jnp.concatenate([Wn0[:, 64:].T, zw], axis=0)

    alo0, ahi0 = _make_sc_agg(128)(x_lo, x_hi, src, dst, z128)
    h_lo, h_hi = _tc_layer0(
        x, alo0, ahi0, Ws0.T, wn0lo, wn0hi, (bs0 + bn0)[None, :])

    alo, ahi = _make_sc_agg(128)(h_lo, h_hi, src, dst, z128)
    h_lo, h_hi = _tc_layer_mid(
        h_lo, h_hi, alo, ahi, alo0, Ws1[:, :128].T, Ws1[:, 128:].T,
        Wn1[:, :128].T, Wn1[:, 128:].T, (bs1 + bn1)[None, :])

    alo, ahi = _make_sc_agg(128)(h_lo, h_hi, src, dst, z128)
    y = _tc_layer_final(
        h_lo, h_hi, alo, ahi, alo0, Ws2[:, :128].T, Ws2[:, 128:].T,
        Wn2[:, :128].T, Wn2[:, 128:].T, (bs2 + bn2)[None, :],
        Wc.T, bc[None, :])
    return y


# speedup vs baseline: 9.4518x; 1.3656x over previous
import functools

import jax
import jax.numpy as jnp
from jax import lax
from jax.experimental import pallas as pl
from jax.experimental.pallas import tpu as pltpu
from jax.experimental.pallas import tpu_sc as plsc

N = 10000
E = 320000
D_IN = 128
D_H = 256
N_CLASSES = 10

NTILES = 16
N_PAD = 10240
ROWS_PER_TILE = N_PAD // NTILES
KB = 64
CHUNK = 16
N_CHUNKS = 20
N_BATCHES = CHUNK * N_CHUNKS
EDGES_PER_TILE = N_BATCHES * KB
E_PAD = EDGES_PER_TILE * NTILES

NBUF = 4

BN = 400
GRID = N // BN


@functools.lru_cache(maxsize=None)
def _make_sc_agg(dh):
    mesh = plsc.VectorSubcoreMesh(core_axis_name="c", subcore_axis_name="s",
                                  num_cores=2, num_subcores=NTILES)

    out_type = [jax.ShapeDtypeStruct((N_PAD, dh), jnp.float32),
                jax.ShapeDtypeStruct((N_PAD, dh), jnp.float32)]
    scratch = [
        pltpu.VMEM((2, CHUNK, KB), jnp.int32),
        pltpu.VMEM((2, CHUNK, KB), jnp.int32),
        pltpu.VMEM((NBUF, KB, dh), jnp.float32),
        pltpu.VMEM_SHARED((N_PAD, dh), jnp.float32),
        pltpu.SemaphoreType.DMA,
        pltpu.SemaphoreType.DMA,
        pltpu.SemaphoreType.DMA,
    ]

    @functools.partial(pl.kernel, mesh=mesh, out_type=out_type,
                       scratch_types=scratch)
    def k(hlo, hhi, src2, dst2, zd, alo, ahi, srcs, dsts, rows, agg,
          gsem, ssem, isem):
        cid = lax.axis_index("c")
        sid = lax.axis_index("s")
        r0 = sid * ROWS_PER_TILE
        bbase = sid * N_BATCHES

        def load_chunk(q, p):
            off = pl.multiple_of(bbase + q * CHUNK, 8)
            pltpu.async_copy(src2.at[pl.ds(off, CHUNK)], srcs.at[p], isem)
            pltpu.async_copy(dst2.at[pl.ds(off, CHUNK)], dsts.at[p], isem)

        def wait_chunk():
            pltpu.make_async_copy(src2.at[pl.ds(0, CHUNK)], srcs.at[0],
                                  isem).wait()
            pltpu.make_async_copy(dst2.at[pl.ds(0, CHUNK)], dsts.at[0],
                                  isem).wait()

        load_chunk(0, 0)
        pltpu.sync_copy(zd.at[pl.ds(r0, ROWS_PER_TILE)],
                        agg.at[pl.ds(r0, ROWS_PER_TILE)])
        wait_chunk()

        plsc.subcore_barrier()

        def edge_loop(h_half):
            def g(b):
                p = lax.rem(b // CHUNK, 2)
                pltpu.async_copy(
                    h_half.at[srcs.at[p, lax.rem(b, CHUNK)]],
                    rows.at[lax.rem(b, NBUF)], gsem)

            def s(b):
                p = lax.rem(b // CHUNK, 2)
                pltpu.async_copy(
                    rows.at[lax.rem(b, NBUF)],
                    agg.at[dsts.at[p, lax.rem(b, CHUNK)]], ssem, add=True)

            def wait_g():
                pltpu.make_async_copy(h_half.at[srcs.at[0, 0]], rows.at[0],
                                      gsem).wait()

            def wait_s():
                pltpu.make_async_copy(rows.at[0], agg.at[dsts.at[0, 0]],
                                      ssem).wait()

            g(0)
            g(1)
            g(2)

            def body(b, c):
                r16 = lax.rem(b, CHUNK)
                wait_g()
                s(b)

                @pl.when(b > 0)
                def _():
                    wait_s()

                @pl.when((r16 == 0) & (b + CHUNK < N_BATCHES))
                def _():
                    q1 = b // CHUNK + 1
                    load_chunk(q1, lax.rem(q1, 2))

                @pl.when((r16 == CHUNK - 3) & (b + 3 < N_BATCHES))
                def _():
                    wait_chunk()

                @pl.when(b + 3 < N_BATCHES)
                def _():
                    g(b + 3)
                return c

            lax.fori_loop(0, N_BATCHES, body, 0)
            wait_s()

        @pl.when(cid == 0)
        def _():
            edge_loop(hlo)

        @pl.when(cid == 1)
        def _():
            edge_loop(hhi)

        plsc.subcore_barrier()

        @pl.when(cid == 0)
        def _():
            pltpu.sync_copy(agg.at[pl.ds(r0, ROWS_PER_TILE)],
                            alo.at[pl.ds(r0, ROWS_PER_TILE)])

        @pl.when(cid == 1)
        def _():
            pltpu.sync_copy(agg.at[pl.ds(r0, ROWS_PER_TILE)],
                            ahi.at[pl.ds(r0, ROWS_PER_TILE)])

    return k


def _dot(a, b):
    return jnp.dot(a, b, preferred_element_type=jnp.float32)


def _row_spec(w):
    return pl.BlockSpec((BN, w), lambda i: (i, 0))


def _full_spec(r, c):
    return pl.BlockSpec((r, c), lambda i: (0, 0))


def _tc_layer0(x, alo, ahi, WsT, WnloT, WnhiT, b):
    def body(x_r, alo_r, ahi_r, ws_r, wnlo_r, wnhi_r, b_r, olo_r, ohi_r):
        hs = _dot(x_r[...], ws_r[...])
        an = _dot(alo_r[...], wnlo_r[...]) + _dot(ahi_r[...], wnhi_r[...])
        d = jnp.maximum(alo_r[...][:, 64:65], 1.0)
        o = jnp.maximum(hs + an / d + b_r[...], 0.0)
        olo_r[...] = o[:, :128]
        ohi_r[...] = o[:, 128:]

    return pl.pallas_call(
        body,
        grid=(GRID,),
        in_specs=[_row_spec(128), _row_spec(128), _row_spec(128),
                  _full_spec(128, 256), _full_spec(128, 256),
                  _full_spec(128, 256), _full_spec(1, 256)],
        out_specs=[_row_spec(128), _row_spec(128)],
        out_shape=[jax.ShapeDtypeStruct((N, 128), jnp.float32),
                   jax.ShapeDtypeStruct((N, 128), jnp.float32)],
    )(x, alo, ahi, WsT, WnloT, WnhiT, b)


def _tc_layer_mid(hlo, hhi, alo, ahi, deg, WsloT, WshiT, WnloT, WnhiT, b):
    def body(hlo_r, hhi_r, alo_r, ahi_r, deg_r, wslo_r, wshi_r, wnlo_r,
             wnhi_r, b_r, olo_r, ohi_r):
        hs = _dot(hlo_r[...], wslo_r[...]) + _dot(hhi_r[...], wshi_r[...])
        an = _dot(alo_r[...], wnlo_r[...]) + _dot(ahi_r[...], wnhi_r[...])
        d = jnp.maximum(deg_r[...][:, 64:65], 1.0)
        o = jnp.maximum(hs + an / d + b_r[...], 0.0)
        olo_r[...] = o[:, :128]
        ohi_r[...] = o[:, 128:]

    return pl.pallas_call(
        body,
        grid=(GRID,),
        in_specs=[_row_spec(128)] * 5 +
                 [_full_spec(128, 256)] * 4 + [_full_spec(1, 256)],
        out_specs=[_row_spec(128), _row_spec(128)],
        out_shape=[jax.ShapeDtypeStruct((N, 128), jnp.float32),
                   jax.ShapeDtypeStruct((N, 128), jnp.float32)],
    )(hlo, hhi, alo, ahi, deg, WsloT, WshiT, WnloT, WnhiT, b)


def _tc_layer_final(hlo, hhi, alo, ahi, deg, WsloT, WshiT, WnloT, WnhiT, b,
                    WcT, bc):
    def body(hlo_r, hhi_r, alo_r, ahi_r, deg_r, wslo_r, wshi_r, wnlo_r,
             wnhi_r, b_r, wct_r, bc_r, cs_r, y_r):
        i = pl.program_id(0)
        hs = _dot(hlo_r[...], wslo_r[...]) + _dot(hhi_r[...], wshi_r[...])
        an = _dot(alo_r[...], wnlo_r[...]) + _dot(ahi_r[...], wnhi_r[...])
        d = jnp.maximum(deg_r[...][:, 64:65], 1.0)
        o = jnp.maximum(hs + an / d + b_r[...], 0.0)
        csum = jnp.sum(o, axis=0, keepdims=True)

        @pl.when(i == 0)
        def _():
            cs_r[...] = csum

        @pl.when(i > 0)
        def _():
            cs_r[...] = cs_r[...] + csum

        @pl.when(i == GRID - 1)
        def _():
            hg = cs_r[...] * (1.0 / N)
            y_r[...] = _dot(hg, wct_r[...]) + bc_r[...]

    csum_out, y = pl.pallas_call(
        body,
        grid=(GRID,),
        in_specs=[_row_spec(128)] * 5 +
                 [_full_spec(128, 256)] * 4 + [_full_spec(1, 256)] +
                 [_full_spec(256, N_CLASSES), _full_spec(1, N_CLASSES)],
        out_specs=[_full_spec(1, 256), _full_spec(1, N_CLASSES)],
        out_shape=[jax.ShapeDtypeStruct((1, 256), jnp.float32),
                   jax.ShapeDtypeStruct((1, N_CLASSES), jnp.float32)],
    )(hlo, hhi, alo, ahi, deg, WsloT, WshiT, WnloT, WnhiT, b, WcT, bc)
    del csum_out
    return y


def kernel(x, edge_index, Ws0, bs0, Wn0, bn0, Ws1, bs1, Wn1, bn1,
           Ws2, bs2, Wn2, bn2, Wc, bc):
    n_pad = E_PAD - E
    src = jnp.concatenate([edge_index[0].astype(jnp.int32),
                           jnp.arange(n_pad, dtype=jnp.int32) % N])
    dst = jnp.concatenate([edge_index[1].astype(jnp.int32),
                           N + (jnp.arange(n_pad, dtype=jnp.int32) % (N_PAD - N))])
    src = src.reshape(NTILES * N_BATCHES, KB)
    dst = dst.reshape(NTILES * N_BATCHES, KB)

    onec = jnp.ones((N, 1), jnp.float32)
    zc = jnp.zeros((N, 63), jnp.float32)
    x_lo = jnp.concatenate([x[:, :64], onec, zc], axis=1)
    x_hi = jnp.concatenate([x[:, 64:], onec, zc], axis=1)

    z128 = jnp.zeros((N_PAD, 128), jnp.float32)

    zw = jnp.zeros((64, 256), jnp.float32)
    wn0lo = jnp.concatenate([Wn0[:, :64].T, zw], axis=0)
    wn0hi = jnp.concatenate([Wn0[:, 64:].T, zw], axis=0)

    alo0, ahi0 = _make_sc_agg(128)(x_lo, x_hi, src, dst, z128)
    h_lo, h_hi = _tc_layer0(
        x, alo0, ahi0, Ws0.T, wn0lo, wn0hi, (bs0 + bn0)[None, :])

    alo, ahi = _make_sc_agg(128)(h_lo, h_hi, src, dst, z128)
    h_lo, h_hi = _tc_layer_mid(
        h_lo, h_hi, alo, ahi, alo0, Ws1[:, :128].T, Ws1[:, 128:].T,
        Wn1[:, :128].T, Wn1[:, 128:].T, (bs1 + bn1)[None, :])

    alo, ahi = _make_sc_agg(128)(h_lo, h_hi, src, dst, z128)
    y = _tc_layer_final(
        h_lo, h_hi, alo, ahi, alo0, Ws2[:, :128].T, Ws2[:, 128:].T,
        Wn2[:, :128].T, Wn2[:, 128:].T, (bs2 + bn2)[None, :],
        Wc.T, bc[None, :])
    return y


# speedup vs baseline: 9.5468x; 1.0101x over previous
import functools

import jax
import jax.numpy as jnp
from jax import lax
from jax.experimental import pallas as pl
from jax.experimental.pallas import tpu as pltpu
from jax.experimental.pallas import tpu_sc as plsc

N = 10000
E = 320000
D_IN = 128
D_H = 256
N_CLASSES = 10

NTILES = 16
N_PAD = 10240
ROWS_PER_TILE = N_PAD // NTILES
KB = 64
CHUNK = 16
N_CHUNKS = 20
N_BATCHES = CHUNK * N_CHUNKS
EDGES_PER_TILE = N_BATCHES * KB
E_PAD = EDGES_PER_TILE * NTILES

NBUF = 5

BN = 400
GRID = N // BN


@functools.lru_cache(maxsize=None)
def _make_sc_agg(dh):
    mesh = plsc.VectorSubcoreMesh(core_axis_name="c", subcore_axis_name="s",
                                  num_cores=2, num_subcores=NTILES)

    out_type = [jax.ShapeDtypeStruct((N_PAD, dh), jnp.float32),
                jax.ShapeDtypeStruct((N_PAD, dh), jnp.float32)]
    scratch = [
        pltpu.VMEM((2, CHUNK, KB), jnp.int32),
        pltpu.VMEM((2, CHUNK, KB), jnp.int32),
        pltpu.VMEM((NBUF, KB, dh), jnp.float32),
        pltpu.VMEM_SHARED((N_PAD, dh), jnp.float32),
        pltpu.SemaphoreType.DMA,
        pltpu.SemaphoreType.DMA,
        pltpu.SemaphoreType.DMA,
    ]

    @functools.partial(pl.kernel, mesh=mesh, out_type=out_type,
                       scratch_types=scratch)
    def k(hlo, hhi, src2, dst2, zd, alo, ahi, srcs, dsts, rows, agg,
          gsem, ssem, isem):
        cid = lax.axis_index("c")
        sid = lax.axis_index("s")
        r0 = sid * ROWS_PER_TILE
        bbase = sid * N_BATCHES

        def load_chunk(q, p):
            off = pl.multiple_of(bbase + q * CHUNK, 8)
            pltpu.async_copy(src2.at[pl.ds(off, CHUNK)], srcs.at[p], isem)
            pltpu.async_copy(dst2.at[pl.ds(off, CHUNK)], dsts.at[p], isem)

        def wait_chunk():
            pltpu.make_async_copy(src2.at[pl.ds(0, CHUNK)], srcs.at[0],
                                  isem).wait()
            pltpu.make_async_copy(dst2.at[pl.ds(0, CHUNK)], dsts.at[0],
                                  isem).wait()

        load_chunk(0, 0)
        pltpu.sync_copy(zd.at[pl.ds(r0, ROWS_PER_TILE)],
                        agg.at[pl.ds(r0, ROWS_PER_TILE)])
        wait_chunk()

        plsc.subcore_barrier()

        def edge_loop(h_half):
            def g(b):
                p = lax.rem(b // CHUNK, 2)
                pltpu.async_copy(
                    h_half.at[srcs.at[p, lax.rem(b, CHUNK)]],
                    rows.at[lax.rem(b, NBUF)], gsem)

            def s(b):
                p = lax.rem(b // CHUNK, 2)
                pltpu.async_copy(
                    rows.at[lax.rem(b, NBUF)],
                    agg.at[dsts.at[p, lax.rem(b, CHUNK)]], ssem, add=True)

            def wait_g():
                pltpu.make_async_copy(h_half.at[srcs.at[0, 0]], rows.at[0],
                                      gsem).wait()

            def wait_s():
                pltpu.make_async_copy(rows.at[0], agg.at[dsts.at[0, 0]],
                                      ssem).wait()

            for k in range(NBUF - 1):
                g(k)

            def body(b, c):
                r16 = lax.rem(b, CHUNK)
                wait_g()
                s(b)

                @pl.when(b > 0)
                def _():
                    wait_s()

                @pl.when((r16 == 0) & (b + CHUNK < N_BATCHES))
                def _():
                    q1 = b // CHUNK + 1
                    load_chunk(q1, lax.rem(q1, 2))

                @pl.when((r16 == CHUNK - (NBUF - 1)) & (b + NBUF - 1 < N_BATCHES))
                def _():
                    wait_chunk()

                @pl.when(b + NBUF - 1 < N_BATCHES)
                def _():
                    g(b + NBUF - 1)
                return c

            lax.fori_loop(0, N_BATCHES, body, 0)
            wait_s()

        @pl.when(cid == 0)
        def _():
            edge_loop(hlo)

        @pl.when(cid == 1)
        def _():
            edge_loop(hhi)

        plsc.subcore_barrier()

        @pl.when(cid == 0)
        def _():
            pltpu.sync_copy(agg.at[pl.ds(r0, ROWS_PER_TILE)],
                            alo.at[pl.ds(r0, ROWS_PER_TILE)])

        @pl.when(cid == 1)
        def _():
            pltpu.sync_copy(agg.at[pl.ds(r0, ROWS_PER_TILE)],
                            ahi.at[pl.ds(r0, ROWS_PER_TILE)])

    return k


def _dot(a, b):
    return jnp.dot(a, b, preferred_element_type=jnp.float32)


def _row_spec(w):
    return pl.BlockSpec((BN, w), lambda i: (i, 0))


def _full_spec(r, c):
    return pl.BlockSpec((r, c), lambda i: (0, 0))


def _tc_layer0(x, alo, ahi, WsT, WnloT, WnhiT, b):
    def body(x_r, alo_r, ahi_r, ws_r, wnlo_r, wnhi_r, b_r, olo_r, ohi_r):
        hs = _dot(x_r[...], ws_r[...])
        an = _dot(alo_r[...], wnlo_r[...]) + _dot(ahi_r[...], wnhi_r[...])
        d = jnp.maximum(alo_r[...][:, 64:65], 1.0)
        o = jnp.maximum(hs + an / d + b_r[...], 0.0)
        olo_r[...] = o[:, :128]
        ohi_r[...] = o[:, 128:]

    return pl.pallas_call(
        body,
        grid=(GRID,),
        in_specs=[_row_spec(128), _row_spec(128), _row_spec(128),
                  _full_spec(128, 256), _full_spec(128, 256),
                  _full_spec(128, 256), _full_spec(1, 256)],
        out_specs=[_row_spec(128), _row_spec(128)],
        out_shape=[jax.ShapeDtypeStruct((N, 128), jnp.float32),
                   jax.ShapeDtypeStruct((N, 128), jnp.float32)],
    )(x, alo, ahi, WsT, WnloT, WnhiT, b)


def _tc_layer_mid(hlo, hhi, alo, ahi, deg, WsloT, WshiT, WnloT, WnhiT, b):
    def body(hlo_r, hhi_r, alo_r, ahi_r, deg_r, wslo_r, wshi_r, wnlo_r,
             wnhi_r, b_r, olo_r, ohi_r):
        hs = _dot(hlo_r[...], wslo_r[...]) + _dot(hhi_r[...], wshi_r[...])
        an = _dot(alo_r[...], wnlo_r[...]) + _dot(ahi_r[...], wnhi_r[...])
        d = jnp.maximum(deg_r[...][:, 64:65], 1.0)
        o = jnp.maximum(hs + an / d + b_r[...], 0.0)
        olo_r[...] = o[:, :128]
        ohi_r[...] = o[:, 128:]

    return pl.pallas_call(
        body,
        grid=(GRID,),
        in_specs=[_row_spec(128)] * 5 +
                 [_full_spec(128, 256)] * 4 + [_full_spec(1, 256)],
        out_specs=[_row_spec(128), _row_spec(128)],
        out_shape=[jax.ShapeDtypeStruct((N, 128), jnp.float32),
                   jax.ShapeDtypeStruct((N, 128), jnp.float32)],
    )(hlo, hhi, alo, ahi, deg, WsloT, WshiT, WnloT, WnhiT, b)


def _tc_layer_final(hlo, hhi, alo, ahi, deg, WsloT, WshiT, WnloT, WnhiT, b,
                    WcT, bc):
    def body(hlo_r, hhi_r, alo_r, ahi_r, deg_r, wslo_r, wshi_r, wnlo_r,
             wnhi_r, b_r, wct_r, bc_r, cs_r, y_r):
        i = pl.program_id(0)
        hs = _dot(hlo_r[...], wslo_r[...]) + _dot(hhi_r[...], wshi_r[...])
        an = _dot(alo_r[...], wnlo_r[...]) + _dot(ahi_r[...], wnhi_r[...])
        d = jnp.maximum(deg_r[...][:, 64:65], 1.0)
        o = jnp.maximum(hs + an / d + b_r[...], 0.0)
        csum = jnp.sum(o, axis=0, keepdims=True)

        @pl.when(i == 0)
        def _():
            cs_r[...] = csum

        @pl.when(i > 0)
        def _():
            cs_r[...] = cs_r[...] + csum

        @pl.when(i == GRID - 1)
        def _():
            hg = cs_r[...] * (1.0 / N)
            y_r[...] = _dot(hg, wct_r[...]) + bc_r[...]

    csum_out, y = pl.pallas_call(
        body,
        grid=(GRID,),
        in_specs=[_row_spec(128)] * 5 +
                 [_full_spec(128, 256)] * 4 + [_full_spec(1, 256)] +
                 [_full_spec(256, N_CLASSES), _full_spec(1, N_CLASSES)],
        out_specs=[_full_spec(1, 256), _full_spec(1, N_CLASSES)],
        out_shape=[jax.ShapeDtypeStruct((1, 256), jnp.float32),
                   jax.ShapeDtypeStruct((1, N_CLASSES), jnp.float32)],
    )(hlo, hhi, alo, ahi, deg, WsloT, WshiT, WnloT, WnhiT, b, WcT, bc)
    del csum_out
    return y


def kernel(x, edge_index, Ws0, bs0, Wn0, bn0, Ws1, bs1, Wn1, bn1,
           Ws2, bs2, Wn2, bn2, Wc, bc):
    n_pad = E_PAD - E
    src = jnp.concatenate([edge_index[0].astype(jnp.int32),
                           jnp.arange(n_pad, dtype=jnp.int32) % N])
    dst = jnp.concatenate([edge_index[1].astype(jnp.int32),
                           N + (jnp.arange(n_pad, dtype=jnp.int32) % (N_PAD - N))])
    src = src.reshape(NTILES * N_BATCHES, KB)
    dst = dst.reshape(NTILES * N_BATCHES, KB)

    onec = jnp.ones((N, 1), jnp.float32)
    zc = jnp.zeros((N, 63), jnp.float32)
    x_lo = jnp.concatenate([x[:, :64], onec, zc], axis=1)
    x_hi = jnp.concatenate([x[:, 64:], onec, zc], axis=1)

    z128 = jnp.zeros((N_PAD, 128), jnp.float32)

    zw = jnp.zeros((64, 256), jnp.float32)
    wn0lo = jnp.concatenate([Wn0[:, :64].T, zw], axis=0)
    wn0hi = jnp.concatenate([Wn0[:, 64:].T, zw], axis=0)

    alo0, ahi0 = _make_sc_agg(128)(x_lo, x_hi, src, dst, z128)
    h_lo, h_hi = _tc_layer0(
        x, alo0, ahi0, Ws0.T, wn0lo, wn0hi, (bs0 + bn0)[None, :])

    alo, ahi = _make_sc_agg(128)(h_lo, h_hi, src, dst, z128)
    h_lo, h_hi = _tc_layer_mid(
        h_lo, h_hi, alo, ahi, alo0, Ws1[:, :128].T, Ws1[:, 128:].T,
        Wn1[:, :128].T, Wn1[:, 128:].T, (bs1 + bn1)[None, :])

    alo, ahi = _make_sc_agg(128)(h_lo, h_hi, src, dst, z128)
    y = _tc_layer_final(
        h_lo, h_hi, alo, ahi, alo0, Ws2[:, :128].T, Ws2[:, 128:].T,
        Wn2[:, :128].T, Wn2[:, 128:].T, (bs2 + bn2)[None, :],
        Wc.T, bc[None, :])
    return y
